# baseline (speedup 1.0000x reference)
import functools

import jax
import jax.numpy as jnp
from jax import lax
from jax.experimental import pallas as pl
from jax.experimental.pallas import tpu as pltpu
from jax.experimental.pallas import tpu_sc as plsc

N_TOK = 16384
DIM = 2048
N_EXP = 64
TB = 1024
NGRP = 4
GTOK = N_TOK // NGRP

NW = 32
L = 16


def _mm_body(x_ref, w_ref, out_ref):
    out_ref[...] = lax.dot_general(
        w_ref[...], x_ref[...],
        dimension_numbers=(((1,), (1,)), ((), ())),
        preferred_element_type=jnp.float32,
    )


_matmul_tc = pl.pallas_call(
    _mm_body,
    grid=(GTOK // TB,),
    in_specs=[
        pl.BlockSpec((TB, DIM), lambda i: (i, 0)),
        pl.BlockSpec((N_EXP, DIM), lambda i: (0, 0)),
    ],
    out_specs=pl.BlockSpec((N_EXP, TB), lambda i: (0, i)),
    out_shape=jax.ShapeDtypeStruct((N_EXP, GTOK), jnp.float32),
)


_sc_mesh = plsc.VectorSubcoreMesh(core_axis_name="c", subcore_axis_name="s")
TPW = GTOK // NW


@functools.partial(
    pl.kernel,
    out_type=[
        jax.ShapeDtypeStruct((GTOK,), jnp.float32),
        jax.ShapeDtypeStruct((GTOK,), jnp.float32),
        jax.ShapeDtypeStruct((GTOK,), jnp.int32),
        jax.ShapeDtypeStruct((GTOK,), jnp.int32),
    ],
    mesh=_sc_mesh,
    scratch_types=[
        pltpu.VMEM((N_EXP, TPW), jnp.float32),
        pltpu.VMEM((TPW,), jnp.float32),
        pltpu.VMEM((TPW,), jnp.float32),
        pltpu.VMEM((TPW,), jnp.int32),
        pltpu.VMEM((TPW,), jnp.int32),
    ],
)
def _topk_sc(logt_hbm, p1_hbm, p2_hbm, i1_hbm, i2_hbm,
             buf, p1v, p2v, i1v, i2v):
    wid = lax.axis_index("s") * 2 + lax.axis_index("c")
    base = wid * TPW
    pltpu.sync_copy(logt_hbm.at[:, pl.ds(base, TPW)], buf)

    def chunk_body(c, carry):
        off = c * L
        neg = jnp.full((L,), -jnp.inf, jnp.float32)
        zero = jnp.zeros((L,), jnp.int32)

        def exp_body(e, st):
            m1, m2, i1, i2 = st
            v = buf[e, pl.ds(off, L)]
            ev = jnp.full((L,), e, jnp.int32)
            gt1 = v > m1
            gt2 = v > m2
            m2n = jnp.where(gt1, m1, jnp.where(gt2, v, m2))
            i2n = jnp.where(gt1, i1, jnp.where(gt2, ev, i2))
            m1n = jnp.where(gt1, v, m1)
            i1n = jnp.where(gt1, ev, i1)
            return m1n, m2n, i1n, i2n

        m1, m2, i1, i2 = lax.fori_loop(
            0, N_EXP, exp_body, (neg, neg, zero, zero))
        e2 = jnp.exp(m2 - m1)
        s = 1.0 + e2
        p1v[pl.ds(off, L)] = 1.0 / s
        p2v[pl.ds(off, L)] = e2 / s
        i1v[pl.ds(off, L)] = i1
        i2v[pl.ds(off, L)] = i2
        return carry

    lax.fori_loop(0, TPW // L, chunk_body, 0)
    pltpu.sync_copy(p1v, p1_hbm.at[pl.ds(base, TPW)])
    pltpu.sync_copy(p2v, p2_hbm.at[pl.ds(base, TPW)])
    pltpu.sync_copy(i1v, i1_hbm.at[pl.ds(base, TPW)])
    pltpu.sync_copy(i2v, i2_hbm.at[pl.ds(base, TPW)])


def kernel(x, W):
    p1s, p2s, i1s, i2s = [], [], [], []
    for g in range(NGRP):
        logt = _matmul_tc(lax.slice_in_dim(x, g * GTOK, (g + 1) * GTOK), W)
        p1, p2, i1, i2 = _topk_sc(logt)
        p1s.append(p1)
        p2s.append(p2)
        i1s.append(i1)
        i2s.append(i2)
    probs = jnp.stack([jnp.concatenate(p1s), jnp.concatenate(p2s)], axis=1)
    idx = jnp.stack([jnp.concatenate(i1s), jnp.concatenate(i2s)], axis=1)
    return probs, idx


# speedup vs baseline: 2.2333x; 2.2333x over previous
import functools

import jax
import jax.numpy as jnp
from jax import lax
from jax.experimental import pallas as pl
from jax.experimental.pallas import tpu as pltpu
from jax.experimental.pallas import tpu_sc as plsc

N_TOK = 16384
DIM = 2048
N_EXP = 64
TB = 2048

NW = 32
TPW = N_TOK // NW
L = 16


def _mm_body(x_ref, w_ref, out_ref):
    out_ref[...] = lax.dot_general(
        w_ref[...], x_ref[...],
        dimension_numbers=(((1,), (1,)), ((), ())),
        preferred_element_type=jnp.float32,
    )


_matmul_tc = pl.pallas_call(
    _mm_body,
    grid=(N_TOK // TB,),
    in_specs=[
        pl.BlockSpec((TB, DIM), lambda i: (i, 0)),
        pl.BlockSpec((N_EXP, DIM), lambda i: (0, 0)),
    ],
    out_specs=pl.BlockSpec((N_EXP, TB), lambda i: (0, i)),
    out_shape=jax.ShapeDtypeStruct((N_EXP, N_TOK), jnp.float32),
)


_sc_mesh = plsc.VectorSubcoreMesh(core_axis_name="c", subcore_axis_name="s")


@functools.partial(
    pl.kernel,
    out_type=[
        jax.ShapeDtypeStruct((N_TOK,), jnp.float32),
        jax.ShapeDtypeStruct((N_TOK,), jnp.float32),
        jax.ShapeDtypeStruct((N_TOK,), jnp.int32),
        jax.ShapeDtypeStruct((N_TOK,), jnp.int32),
    ],
    mesh=_sc_mesh,
    scratch_types=[
        pltpu.VMEM((N_EXP, TPW), jnp.float32),
        pltpu.VMEM((TPW,), jnp.float32),
        pltpu.VMEM((TPW,), jnp.float32),
        pltpu.VMEM((TPW,), jnp.int32),
        pltpu.VMEM((TPW,), jnp.int32),
    ],
)
def _topk_sc(logt_hbm, p1_hbm, p2_hbm, i1_hbm, i2_hbm,
             buf, p1v, p2v, i1v, i2v):
    wid = lax.axis_index("s") * 2 + lax.axis_index("c")
    base = wid * TPW
    pltpu.sync_copy(logt_hbm.at[:, pl.ds(base, TPW)], buf)

    def chunk_body(c, carry):
        off = c * L
        m1 = buf[0, pl.ds(off, L)]
        i1 = jnp.zeros((L,), jnp.int32)
        m2 = jnp.full((L,), -jnp.inf, jnp.float32)
        i2 = jnp.zeros((L,), jnp.int32)
        for e in range(1, N_EXP):
            v = buf[e, pl.ds(off, L)]
            ev = jnp.full((L,), e, jnp.int32)
            gt1 = v > m1
            gt2 = v > m2
            m2 = jnp.where(gt1, m1, jnp.where(gt2, v, m2))
            i2 = jnp.where(gt1, i1, jnp.where(gt2, ev, i2))
            m1 = jnp.where(gt1, v, m1)
            i1 = jnp.where(gt1, ev, i1)
        e2 = jnp.exp(m2 - m1)
        s = 1.0 + e2
        p1v[pl.ds(off, L)] = 1.0 / s
        p2v[pl.ds(off, L)] = e2 / s
        i1v[pl.ds(off, L)] = i1
        i2v[pl.ds(off, L)] = i2
        return carry

    lax.fori_loop(0, TPW // L, chunk_body, 0)
    pltpu.sync_copy(p1v, p1_hbm.at[pl.ds(base, TPW)])
    pltpu.sync_copy(p2v, p2_hbm.at[pl.ds(base, TPW)])
    pltpu.sync_copy(i1v, i1_hbm.at[pl.ds(base, TPW)])
    pltpu.sync_copy(i2v, i2_hbm.at[pl.ds(base, TPW)])


def kernel(x, W):
    logt = _matmul_tc(x, W)
    p1, p2, i1, i2 = _topk_sc(logt)
    probs = jnp.stack([p1, p2], axis=1)
    idx = jnp.stack([i1, i2], axis=1)
    return probs, idx


# speedup vs baseline: 2.2796x; 1.0208x over previous
import functools

import jax
import jax.numpy as jnp
from jax import lax
from jax.experimental import pallas as pl
from jax.experimental.pallas import tpu as pltpu
from jax.experimental.pallas import tpu_sc as plsc

N_TOK = 16384
DIM = 2048
N_EXP = 64
TB = 2048

NW = 32
TPW = N_TOK // NW
L = 16


def _mm_body(x_ref, w_ref, out_ref):
    out_ref[...] = lax.dot_general(
        w_ref[...], x_ref[...],
        dimension_numbers=(((1,), (1,)), ((), ())),
        preferred_element_type=jnp.float32,
    )


_matmul_tc = pl.pallas_call(
    _mm_body,
    grid=(N_TOK // TB,),
    in_specs=[
        pl.BlockSpec((TB, DIM), lambda i: (i, 0)),
        pl.BlockSpec((N_EXP, DIM), lambda i: (0, 0)),
    ],
    out_specs=pl.BlockSpec((N_EXP, TB), lambda i: (0, i)),
    out_shape=jax.ShapeDtypeStruct((N_EXP, N_TOK), jnp.float32),
)


_sc_mesh = plsc.VectorSubcoreMesh(core_axis_name="c", subcore_axis_name="s")


@functools.partial(
    pl.kernel,
    out_type=[
        jax.ShapeDtypeStruct((N_TOK,), jnp.float32),
        jax.ShapeDtypeStruct((N_TOK,), jnp.float32),
        jax.ShapeDtypeStruct((N_TOK,), jnp.int32),
        jax.ShapeDtypeStruct((N_TOK,), jnp.int32),
    ],
    mesh=_sc_mesh,
    scratch_types=[
        pltpu.VMEM((N_EXP, TPW), jnp.float32),
        pltpu.VMEM((TPW,), jnp.float32),
        pltpu.VMEM((TPW,), jnp.float32),
        pltpu.VMEM((TPW,), jnp.int32),
        pltpu.VMEM((TPW,), jnp.int32),
    ],
)
def _topk_sc(logt_hbm, p1_hbm, p2_hbm, i1_hbm, i2_hbm,
             buf, p1v, p2v, i1v, i2v):
    wid = lax.axis_index("s") * 2 + lax.axis_index("c")
    base = wid * TPW
    pltpu.sync_copy(logt_hbm.at[:, pl.ds(base, TPW)], buf)

    def chunk_body(c, carry):
        offs = (c * (2 * L), c * (2 * L) + L)
        st = []
        for off in offs:
            m1 = buf[0, pl.ds(off, L)]
            st.append([m1, jnp.full((L,), -jnp.inf, jnp.float32),
                       jnp.zeros((L,), jnp.int32), jnp.zeros((L,), jnp.int32)])
        for e in range(1, N_EXP):
            ev = jnp.full((L,), e, jnp.int32)
            for off, s in zip(offs, st):
                m1, m2, i1, i2 = s
                v = buf[e, pl.ds(off, L)]
                gt1 = v > m1
                gt2 = v > m2
                s[3] = jnp.where(gt1, i1, jnp.where(gt2, ev, i2))
                s[2] = jnp.where(gt1, ev, i1)
                s[1] = jnp.maximum(m2, jnp.minimum(m1, v))
                s[0] = jnp.maximum(m1, v)
        for off, (m1, m2, i1, i2) in zip(offs, st):
            e2 = jnp.exp(m2 - m1)
            den = 1.0 + e2
            p1v[pl.ds(off, L)] = 1.0 / den
            p2v[pl.ds(off, L)] = e2 / den
            i1v[pl.ds(off, L)] = i1
            i2v[pl.ds(off, L)] = i2
        return carry

    lax.fori_loop(0, TPW // (2 * L), chunk_body, 0)
    pltpu.sync_copy(p1v, p1_hbm.at[pl.ds(base, TPW)])
    pltpu.sync_copy(p2v, p2_hbm.at[pl.ds(base, TPW)])
    pltpu.sync_copy(i1v, i1_hbm.at[pl.ds(base, TPW)])
    pltpu.sync_copy(i2v, i2_hbm.at[pl.ds(base, TPW)])


def kernel(x, W):
    logt = _matmul_tc(x, W)
    p1, p2, i1, i2 = _topk_sc(logt)
    probs = jnp.stack([p1, p2], axis=1)
    idx = jnp.stack([i1, i2], axis=1)
    return probs, idx
